# Initial kernel scaffold; baseline (speedup 1.0000x reference)
import jax
import jax.numpy as jnp
from jax.experimental import pallas as pl


def kernel(cosine, label):
    raise NotImplementedError("write your pallas kernel here")


# speedup vs baseline: 4.4515x; 4.4515x over previous
import functools
import math

import jax
import jax.numpy as jnp
from jax.experimental import pallas as pl
from jax.experimental.pallas import tpu as pltpu

_MARGIN = 0.1
_SCALE = 64.0
_COS_M = math.cos(_MARGIN)
_SIN_M = math.sin(_MARGIN)

_CB = 2048


def _arcface_tc_body(cos_ref, lab_ref, out_ref, acc_ref, gacc_ref, *, B, C, n_blocks):
    i = pl.program_id(0)

    @pl.when(i == 0)
    def _init():
        acc_ref[...] = jnp.zeros_like(acc_ref)
        gacc_ref[...] = jnp.zeros_like(gacc_ref)

    c = cos_ref[...]
    col = jax.lax.broadcasted_iota(jnp.int32, (B, _CB), 1) + i * _CB
    lab = lab_ref[...]
    safe_lab = jnp.where(lab < 0, 0, lab)

    e = jnp.where(col < C, jnp.exp(c * _SCALE - _SCALE), 0.0)
    acc_ref[...] += jnp.sum(e.reshape(B, _CB // 128, 128), axis=1)

    sel = jnp.where(col == safe_lab, c, 0.0)
    gacc_ref[...] += jnp.sum(sel.reshape(B, _CB // 128, 128), axis=1)

    @pl.when(i == n_blocks - 1)
    def _finish():
        S = jnp.sum(acc_ref[...], axis=1, keepdims=True)
        g = jnp.sum(gacc_ref[...], axis=1, keepdims=True)
        lab2 = lab_ref[...]
        valid = lab2 >= 0
        o = _SCALE * g
        sin_t = jnp.sqrt(jnp.maximum(1.0 - g * g, 0.0))
        v = jnp.where(valid, _SCALE * (_COS_M * g - _SIN_M * sin_t), o)
        S_corr = S - jnp.exp(o - _SCALE) + jnp.exp(v - _SCALE)
        loss_i = _SCALE - v + jnp.log(S_corr)
        out_ref[...] = jnp.sum(loss_i, axis=0, keepdims=True) / B


def kernel(cosine, label):
    B, C = cosine.shape
    label = label.astype(jnp.int32).reshape(B, 1)
    n_blocks = pl.cdiv(C, _CB)

    out = pl.pallas_call(
        functools.partial(_arcface_tc_body, B=B, C=C, n_blocks=n_blocks),
        grid=(n_blocks,),
        in_specs=[
            pl.BlockSpec((B, _CB), lambda i: (0, i)),
            pl.BlockSpec((B, 1), lambda i: (0, 0)),
        ],
        out_specs=pl.BlockSpec((1, 1), lambda i: (0, 0)),
        out_shape=jax.ShapeDtypeStruct((1, 1), jnp.float32),
        scratch_shapes=[
            pltpu.VMEM((B, 128), jnp.float32),
            pltpu.VMEM((B, 128), jnp.float32),
        ],
    )(cosine, label)
    return out[0, 0]


# speedup vs baseline: 6.5647x; 1.4747x over previous
import functools
import math

import jax
import jax.numpy as jnp
from jax.experimental import pallas as pl
from jax.experimental.pallas import tpu as pltpu

_MARGIN = 0.1
_SCALE = 64.0
_COS_M = math.cos(_MARGIN)
_SIN_M = math.sin(_MARGIN)

_CB = 2048


def _arcface_tc_body(cos_ref, lab_ref, out_ref, acc_ref, gacc_ref, *, B, C, n_blocks):
    i = pl.program_id(0)

    @pl.when(i == 0)
    def _init():
        acc_ref[...] = jnp.zeros_like(acc_ref)
        gacc_ref[...] = jnp.zeros_like(gacc_ref)

    c = cos_ref[...]
    col = jax.lax.broadcasted_iota(jnp.int32, (B, _CB), 1) + i * _CB
    lab = lab_ref[...]
    safe_lab = jnp.where(lab < 0, 0, lab)

    n_full = C // _CB

    @pl.when(i < n_full)
    def _main():
        acc_ref[...] += jnp.exp(c * _SCALE - _SCALE)

    @pl.when(i >= n_full)
    def _tail():
        acc_ref[...] += jnp.where(col < C, jnp.exp(c * _SCALE - _SCALE), 0.0)

    gacc_ref[...] += jnp.where(col == safe_lab, c, 0.0)

    @pl.when(i == n_blocks - 1)
    def _finish():
        S = jnp.sum(acc_ref[...], axis=1, keepdims=True)
        g = jnp.sum(gacc_ref[...], axis=1, keepdims=True)
        lab2 = lab_ref[...]
        valid = lab2 >= 0
        o = _SCALE * g
        sin_t = jnp.sqrt(jnp.maximum(1.0 - g * g, 0.0))
        v = jnp.where(valid, _SCALE * (_COS_M * g - _SIN_M * sin_t), o)
        S_corr = S - jnp.exp(o - _SCALE) + jnp.exp(v - _SCALE)
        loss_i = _SCALE - v + jnp.log(S_corr)
        out_ref[...] = jnp.sum(loss_i, axis=0, keepdims=True) / B


def kernel(cosine, label):
    B, C = cosine.shape
    label = label.astype(jnp.int32).reshape(B, 1)
    n_blocks = pl.cdiv(C, _CB)

    out = pl.pallas_call(
        functools.partial(_arcface_tc_body, B=B, C=C, n_blocks=n_blocks),
        grid=(n_blocks,),
        in_specs=[
            pl.BlockSpec((B, _CB), lambda i: (0, i)),
            pl.BlockSpec((B, 1), lambda i: (0, 0)),
        ],
        out_specs=pl.BlockSpec((1, 1), lambda i: (0, 0)),
        out_shape=jax.ShapeDtypeStruct((1, 1), jnp.float32),
        scratch_shapes=[
            pltpu.VMEM((B, _CB), jnp.float32),
            pltpu.VMEM((B, _CB), jnp.float32),
        ],
    )(cosine, label)
    return out[0, 0]


# speedup vs baseline: 20.0178x; 3.0493x over previous
import functools
import math

import jax
import jax.numpy as jnp
from jax.experimental import pallas as pl
from jax.experimental.pallas import tpu as pltpu

_MARGIN = 0.1
_SCALE = 64.0
_COS_M = math.cos(_MARGIN)
_SIN_M = math.sin(_MARGIN)

_RB = 2048


def _arcface_tc_body(ct_ref, lab_ref, out_ref, acc_ref, gacc_ref, *, B, C, n_blocks):
    i = pl.program_id(0)

    @pl.when(i == 0)
    def _init():
        acc_ref[...] = jnp.zeros_like(acc_ref)
        gacc_ref[...] = jnp.zeros_like(gacc_ref)

    c = ct_ref[...]
    row = jax.lax.broadcasted_iota(jnp.int32, (_RB, B), 0) + i * _RB
    lab = lab_ref[...]
    safe_lab = jnp.where(lab < 0, 0, lab)

    n_full = C // _RB

    @pl.when(i < n_full)
    def _main():
        e = jnp.exp(c * _SCALE - _SCALE)
        acc_ref[...] += jnp.sum(e.reshape(_RB // 8, 8, B), axis=0)

    @pl.when(i >= n_full)
    def _tail():
        e = jnp.where(row < C, jnp.exp(c * _SCALE - _SCALE), 0.0)
        acc_ref[...] += jnp.sum(e.reshape(_RB // 8, 8, B), axis=0)

    sel = jnp.where(row == safe_lab, c, 0.0)
    gacc_ref[...] += jnp.sum(sel.reshape(_RB // 8, 8, B), axis=0)

    @pl.when(i == n_blocks - 1)
    def _finish():
        S = jnp.sum(acc_ref[...], axis=0, keepdims=True)
        g = jnp.sum(gacc_ref[...], axis=0, keepdims=True)
        lab2 = lab_ref[...]
        valid = lab2 >= 0
        o = _SCALE * g
        sin_t = jnp.sqrt(jnp.maximum(1.0 - g * g, 0.0))
        v = jnp.where(valid, _SCALE * (_COS_M * g - _SIN_M * sin_t), o)
        S_corr = S - jnp.exp(o - _SCALE) + jnp.exp(v - _SCALE)
        loss_i = _SCALE - v + jnp.log(S_corr)
        out_ref[...] = jnp.sum(loss_i, axis=1, keepdims=True) / B


def kernel(cosine, label):
    B, C = cosine.shape
    ct = cosine.T
    lab = label.astype(jnp.int32).reshape(1, B)
    n_blocks = pl.cdiv(C, _RB)

    out = pl.pallas_call(
        functools.partial(_arcface_tc_body, B=B, C=C, n_blocks=n_blocks),
        grid=(n_blocks,),
        in_specs=[
            pl.BlockSpec((_RB, B), lambda i: (i, 0)),
            pl.BlockSpec((1, B), lambda i: (0, 0)),
        ],
        out_specs=pl.BlockSpec((1, 1), lambda i: (0, 0)),
        out_shape=jax.ShapeDtypeStruct((1, 1), jnp.float32),
        scratch_shapes=[
            pltpu.VMEM((8, B), jnp.float32),
            pltpu.VMEM((8, B), jnp.float32),
        ],
    )(ct, lab)
    return out[0, 0]


# speedup vs baseline: 20.9938x; 1.0488x over previous
import functools
import math

import jax
import jax.numpy as jnp
from jax.experimental import pallas as pl
from jax.experimental.pallas import tpu as pltpu

_MARGIN = 0.1
_SCALE = 64.0
_K2 = _SCALE * math.log2(math.e)
_COS_M = math.cos(_MARGIN)
_SIN_M = math.sin(_MARGIN)

_RB = 2048


def _arcface_tc_body(ct_ref, lab_ref, out_ref, acc_ref, gacc_ref, *, B, C, n_blocks):
    i = pl.program_id(0)

    @pl.when(i == 0)
    def _init():
        acc_ref[...] = jnp.zeros_like(acc_ref)
        gacc_ref[...] = jnp.zeros_like(gacc_ref)

    c = ct_ref[...]
    row = jax.lax.broadcasted_iota(jnp.int32, (_RB, B), 0) + i * _RB
    lab = lab_ref[...]
    safe_lab = jnp.where(lab < 0, 0, lab)

    n_full = C // _RB

    @pl.when(i < n_full)
    def _main():
        e = jnp.exp2(c * _K2 - _K2)
        acc_ref[...] += jnp.sum(e.reshape(_RB // 8, 8, B), axis=0)

    @pl.when(i >= n_full)
    def _tail():
        e = jnp.where(row < C, jnp.exp2(c * _K2 - _K2), 0.0)
        acc_ref[...] += jnp.sum(e.reshape(_RB // 8, 8, B), axis=0)

    sel = jnp.where(row == safe_lab, c, 0.0)
    gacc_ref[...] += jnp.sum(sel.reshape(_RB // 8, 8, B), axis=0)

    @pl.when(i == n_blocks - 1)
    def _finish():
        S = jnp.sum(acc_ref[...], axis=0, keepdims=True)
        g = jnp.sum(gacc_ref[...], axis=0, keepdims=True)
        lab2 = lab_ref[...]
        valid = lab2 >= 0
        o = _SCALE * g
        sin_t = jnp.sqrt(jnp.maximum(1.0 - g * g, 0.0))
        v = jnp.where(valid, _SCALE * (_COS_M * g - _SIN_M * sin_t), o)
        S_corr = S - jnp.exp(o - _SCALE) + jnp.exp(v - _SCALE)
        loss_i = _SCALE - v + jnp.log(S_corr)
        out_ref[...] = jnp.sum(loss_i, axis=1, keepdims=True) / B


def kernel(cosine, label):
    B, C = cosine.shape
    ct = cosine.T
    lab = label.astype(jnp.int32).reshape(1, B)
    n_blocks = pl.cdiv(C, _RB)

    out = pl.pallas_call(
        functools.partial(_arcface_tc_body, B=B, C=C, n_blocks=n_blocks),
        grid=(n_blocks,),
        in_specs=[
            pl.BlockSpec((_RB, B), lambda i: (i, 0)),
            pl.BlockSpec((1, B), lambda i: (0, 0)),
        ],
        out_specs=pl.BlockSpec((1, 1), lambda i: (0, 0)),
        out_shape=jax.ShapeDtypeStruct((1, 1), jnp.float32),
        scratch_shapes=[
            pltpu.VMEM((8, B), jnp.float32),
            pltpu.VMEM((8, B), jnp.float32),
        ],
    )(ct, lab)
    return out[0, 0]
